# baseline (device time: 28779 ns/iter reference)
import functools

import jax
import jax.numpy as jnp
from jax import lax
from jax.experimental import pallas as pl
from jax.experimental.pallas import tpu as pltpu

N_DEV = 16
B = 2
SQ = 128
H_LOC = 4
DH = 64
D_MODEL = 512
ROWS = B * SQ
CH = ROWS // N_DEV


def kernel(x, Wq, K_ext, V_ext, Wo):
    my = lax.axis_index("i")
    k_loc = lax.dynamic_slice_in_dim(K_ext, my * H_LOC, H_LOC, axis=2)
    v_loc = lax.dynamic_slice_in_dim(V_ext, my * H_LOC, H_LOC, axis=2)
    k_loc = jnp.transpose(k_loc, (0, 2, 1, 3))
    v_loc = jnp.transpose(v_loc, (0, 2, 1, 3))

    def body(x_ref, wq_ref, k_ref, v_ref, wo_ref, out_ref,
             partial_ref, rs_buf, send1, recv1, send2, recv2):
        my_pos = lax.axis_index("i")

        barrier_sem = pltpu.get_barrier_semaphore()
        for j in range(1, N_DEV):
            tgt = lax.rem(my_pos + j, N_DEV)
            pl.semaphore_signal(
                barrier_sem, inc=1,
                device_id=(tgt,), device_id_type=pl.DeviceIdType.MESH,
            )
        pl.semaphore_wait(barrier_sem, N_DEV - 1)

        for b in range(B):
            qb = jnp.dot(x_ref[b], wq_ref[...],
                         preferred_element_type=jnp.float32)
            acc = jnp.zeros((SQ, D_MODEL), jnp.float32)
            for hh in range(H_LOC):
                q = qb[:, hh * DH:(hh + 1) * DH]
                k = k_ref[b, hh]
                v = v_ref[b, hh]
                s = lax.dot_general(
                    q, k, (((1,), (1,)), ((), ())),
                    preferred_element_type=jnp.float32) * 0.125
                m = jnp.max(s, axis=1, keepdims=True)
                w = jnp.exp(s - m)
                w = w / jnp.sum(w, axis=1, keepdims=True)
                ctx = jnp.dot(w, v, preferred_element_type=jnp.float32)
                acc = acc + jnp.dot(
                    ctx, wo_ref[hh * DH:(hh + 1) * DH, :],
                    preferred_element_type=jnp.float32)
            partial_ref[pl.ds(b * SQ, SQ), :] = acc

        p1 = []
        for j in range(1, N_DEV):
            tgt = lax.rem(my_pos + j, N_DEV)
            rdma = pltpu.make_async_remote_copy(
                src_ref=partial_ref.at[pl.ds(tgt * CH, CH), :],
                dst_ref=rs_buf.at[j - 1],
                send_sem=send1.at[j - 1],
                recv_sem=recv1.at[j - 1],
                device_id=(tgt,),
                device_id_type=pl.DeviceIdType.MESH,
            )
            rdma.start()
            p1.append(rdma)

        for rdma in p1:
            rdma.wait_recv()
        chunk = partial_ref[pl.ds(my_pos * CH, CH), :] + jnp.sum(
            rs_buf[...], axis=0)
        out_ref[pl.ds(my_pos * CH, CH), :] = chunk

        p2 = []
        for j in range(1, N_DEV):
            tgt = lax.rem(my_pos + j, N_DEV)
            rdma = pltpu.make_async_remote_copy(
                src_ref=out_ref.at[pl.ds(my_pos * CH, CH), :],
                dst_ref=out_ref.at[pl.ds(my_pos * CH, CH), :],
                send_sem=send2.at[j - 1],
                recv_sem=recv2.at[j - 1],
                device_id=(tgt,),
                device_id_type=pl.DeviceIdType.MESH,
            )
            rdma.start()
            p2.append(rdma)

        for rdma in p1:
            rdma.wait_send()
        for rdma in p2:
            rdma.wait_send()
        for rdma in p2:
            rdma.wait_recv()

    out = pl.pallas_call(
        body,
        out_shape=jax.ShapeDtypeStruct((ROWS, D_MODEL), jnp.float32),
        in_specs=[pl.BlockSpec(memory_space=pltpu.VMEM)] * 5,
        out_specs=pl.BlockSpec(memory_space=pltpu.VMEM),
        scratch_shapes=[
            pltpu.VMEM((ROWS, D_MODEL), jnp.float32),
            pltpu.VMEM((N_DEV - 1, CH, D_MODEL), jnp.float32),
            pltpu.SemaphoreType.DMA((N_DEV - 1,)),
            pltpu.SemaphoreType.DMA((N_DEV - 1,)),
            pltpu.SemaphoreType.DMA((N_DEV - 1,)),
            pltpu.SemaphoreType.DMA((N_DEV - 1,)),
        ],
        compiler_params=pltpu.CompilerParams(collective_id=0),
    )(x, Wq, k_loc, v_loc, Wo)

    return out.reshape(B, SQ, D_MODEL)
